# baseline (speedup 1.0000x reference)
import dataclasses
import functools

import jax
import jax.numpy as jnp
from jax import lax
from jax.experimental import pallas as pl
from jax.experimental.pallas import tpu as pltpu
from jax.experimental.pallas import tpu_sc as plsc

N = 10000
E = 320000
F_IN = 128
F_HID = 256
F_OUT = 64

NC = 2
NS = 16
LANES = 16
EPT = E // NS
CHUNK = 125
NCHUNK = EPT // CHUNK
ROWS_PER_TILE = N // NS
NB = 2000
DEPTH = 4


def _sc_compiler_params(untiled=False):
    cp = pltpu.CompilerParams()
    if "needs_layout_passes" in pltpu.CompilerParams.__dataclass_fields__:
        cp = dataclasses.replace(cp, needs_layout_passes=False)
    if untiled and "use_tc_tiling_on_sc" in pltpu.CompilerParams.__dataclass_fields__:
        cp = dataclasses.replace(cp, use_tc_tiling_on_sc=False)
    return cp


def _sc_degree_hists(edges_r):
    mesh = plsc.VectorSubcoreMesh(core_axis_name="c", subcore_axis_name="s")

    @functools.partial(
        pl.kernel,
        out_type=jax.ShapeDtypeStruct((NC, NS, N), jnp.float32),
        mesh=mesh,
        compiler_params=_sc_compiler_params(),
        scratch_types=[
            pltpu.VMEM((EPT,), jnp.int32),
            pltpu.VMEM((N,), jnp.float32),
        ],
    )
    def k(edges_hbm, out_hbm, ebuf, hist):
        c = lax.axis_index("c")
        s = lax.axis_index("s")
        pltpu.sync_copy(edges_hbm.at[c, s], ebuf)
        zeros16 = jnp.zeros((LANES,), jnp.float32)
        ones16 = jnp.ones((LANES,), jnp.float32)

        @pl.loop(0, N // LANES)
        def _(i):
            hist[pl.ds(i * LANES, LANES)] = zeros16

        @pl.loop(0, EPT // LANES)
        def _(i):
            idx = ebuf[pl.ds(i * LANES, LANES)]
            plsc.addupdate_scatter(hist, [idx], ones16)

        pltpu.sync_copy(hist, out_hbm.at[c, s])

    return k(edges_r)


def _sc_propagate(xblocks, srcr, dstr, fb):
    nblk = xblocks.shape[0]
    per_core = nblk // NC
    mesh = plsc.VectorSubcoreMesh(core_axis_name="c", subcore_axis_name="s")

    @functools.partial(
        pl.kernel,
        out_type=jax.ShapeDtypeStruct((nblk, NS, ROWS_PER_TILE, fb),
                                      jnp.float32),
        mesh=mesh,
        compiler_params=_sc_compiler_params(untiled=True),
        scratch_types=(
            [pltpu.VMEM((NCHUNK, CHUNK), jnp.int32),
             pltpu.VMEM((NCHUNK, CHUNK), jnp.int32)]
            + [pltpu.VMEM((CHUNK, fb), jnp.float32)] * (DEPTH + 1)
            + [pltpu.VMEM_SHARED((N, fb), jnp.float32)]
            + [pltpu.SemaphoreType.DMA] * (2 * DEPTH)
        ),
    )
    def k(*refs):
        xb_hbm, src_hbm, dst_hbm, out_hbm, src_v, dst_v = refs[:6]
        gbufs = refs[6:6 + DEPTH]
        zbuf = refs[6 + DEPTH]
        acc = refs[7 + DEPTH]
        sgs = refs[8 + DEPTH:8 + 2 * DEPTH]
        sss = refs[8 + 2 * DEPTH:8 + 3 * DEPTH]
        x_hbms = [xb_hbm.at[b] for b in range(nblk)]
        c = lax.axis_index("c")
        s = lax.axis_index("s")
        zeros16 = jnp.zeros((LANES,), jnp.float32)

        @pl.loop(0, CHUNK)
        def _(r):
            @pl.loop(0, fb // LANES)
            def _(kk):
                zbuf[r, pl.ds(kk * LANES, LANES)] = zeros16

        pltpu.sync_copy(src_hbm.at[s], src_v)
        pltpu.sync_copy(dst_hbm.at[s], dst_v)

        def run_pass(x_hbm, blk):
            for q in range(ROWS_PER_TILE // CHUNK):
                pltpu.sync_copy(
                    zbuf, acc.at[pl.ds(s * ROWS_PER_TILE + q * CHUNK, CHUNK)])
            plsc.subcore_barrier()

            def wait_g(b):
                pltpu.make_async_copy(x_hbm.at[src_v.at[0]], gbufs[b],
                                      sgs[b]).wait()

            def wait_s(b):
                pltpu.make_async_copy(gbufs[b], acc.at[dst_v.at[0]],
                                      sss[b]).wait()

            for b in range(DEPTH // 2):
                pltpu.async_copy(x_hbm.at[src_v.at[b]], gbufs[b], sgs[b])

            @pl.loop(0, NCHUNK // DEPTH)
            def _(jj):
                for r in range(DEPTH):
                    j = DEPTH * jj + r
                    bn = (r + DEPTH // 2) % DEPTH

                    def step1(bn=bn):
                        wait_s(bn)

                    def step2(j=j, bn=bn):
                        pltpu.async_copy(x_hbm.at[src_v.at[j + DEPTH // 2]],
                                         gbufs[bn], sgs[bn])

                    if r < DEPTH // 2:
                        @pl.when(jj > 0)
                        def _(step1=step1):
                            step1()

                        step2()
                    else:
                        step1()

                        @pl.when(jj < NCHUNK // DEPTH - 1)
                        def _(step2=step2):
                            step2()

                    wait_g(r)
                    pltpu.async_copy(gbufs[r], acc.at[dst_v.at[j]], sss[r],
                                     add=True)

            for b in range(DEPTH // 2, DEPTH):
                wait_s(b)
            plsc.subcore_barrier()
            pltpu.sync_copy(acc.at[pl.ds(s * ROWS_PER_TILE, ROWS_PER_TILE)],
                            out_hbm.at[blk, s])

        for p in range(per_core):
            @pl.when(c == 0)
            def _():
                run_pass(x_hbms[p], p)

            @pl.when(c == 1)
            def _():
                run_pass(x_hbms[per_core + p], per_core + p)

    return k(xblocks, srcr, dstr).reshape(nblk, N, fb)


def _tc_norms_scale(hists, features):
    def body(h_ref, f_ref, ns_ref, nd_ref, xs_ref):
        deg = jnp.sum(h_ref[...], axis=1)
        norm = jnp.where(deg > 0, lax.rsqrt(jnp.maximum(deg, 1.0)), 0.0)
        ns = norm[0][:, None]
        nd = norm[1][:, None]
        ns_ref[...] = ns
        nd_ref[...] = nd
        xsc = f_ref[...] * ns
        xs_ref[0] = xsc[:, : F_IN // 2]
        xs_ref[1] = xsc[:, F_IN // 2:]

    return pl.pallas_call(
        body,
        out_shape=[
            jax.ShapeDtypeStruct((N, 1), jnp.float32),
            jax.ShapeDtypeStruct((N, 1), jnp.float32),
            jax.ShapeDtypeStruct((2, N, F_IN // 2), jnp.float32),
        ],
    )(hists, features)


def _tc_layer1(agg, nd, ns, W1, b1):
    def body(a_ref, nd_ref, ns_ref, w_ref, b_ref, o_ref):
        a = jnp.concatenate([a_ref[0], a_ref[1]], axis=1) * nd_ref[...]
        h = jnp.dot(a, w_ref[...], precision=lax.Precision.HIGHEST) + b_ref[...]
        h = jnp.maximum(h, 0.0) * ns_ref[...]
        for b in range(4):
            o_ref[b] = h[:, b * (F_HID // 4): (b + 1) * (F_HID // 4)]

    return pl.pallas_call(
        body,
        grid=(N // NB,),
        in_specs=[
            pl.BlockSpec((2, NB, F_IN // 2), lambda i: (0, i, 0)),
            pl.BlockSpec((NB, 1), lambda i: (i, 0)),
            pl.BlockSpec((NB, 1), lambda i: (i, 0)),
            pl.BlockSpec((F_IN, F_HID), lambda i: (0, 0)),
            pl.BlockSpec((1, F_HID), lambda i: (0, 0)),
        ],
        out_specs=pl.BlockSpec((4, NB, F_HID // 4), lambda i: (0, i, 0)),
        out_shape=jax.ShapeDtypeStruct((4, N, F_HID // 4), jnp.float32),
    )(agg, nd, ns, W1, b1)


def _tc_layer2(agg, nd, ns, W2, b2, W3):
    def body(a_ref, nd_ref, ns_ref, w2_ref, b2_ref, w3_ref, o_ref):
        a = jnp.concatenate([a_ref[b] for b in range(4)], axis=1) * nd_ref[...]
        h = jnp.dot(a, w2_ref[...], precision=lax.Precision.HIGHEST) + b2_ref[...]
        h = jnp.maximum(h, 0.0) * ns_ref[...]
        t = jnp.dot(h, w3_ref[...], precision=lax.Precision.HIGHEST)
        o_ref[0] = t[:, : F_OUT // 2]
        o_ref[1] = t[:, F_OUT // 2:]

    return pl.pallas_call(
        body,
        grid=(N // NB,),
        in_specs=[
            pl.BlockSpec((4, NB, F_HID // 4), lambda i: (0, i, 0)),
            pl.BlockSpec((NB, 1), lambda i: (i, 0)),
            pl.BlockSpec((NB, 1), lambda i: (i, 0)),
            pl.BlockSpec((F_HID, F_HID), lambda i: (0, 0)),
            pl.BlockSpec((1, F_HID), lambda i: (0, 0)),
            pl.BlockSpec((F_HID, F_OUT), lambda i: (0, 0)),
        ],
        out_specs=pl.BlockSpec((2, NB, F_OUT // 2), lambda i: (0, i, 0)),
        out_shape=jax.ShapeDtypeStruct((2, N, F_OUT // 2), jnp.float32),
    )(agg, nd, ns, W2, b2, W3)


def _tc_final(agg, nd, b3, features):
    def body(a_ref, nd_ref, b_ref, f_ref, o_ref):
        o = jnp.concatenate([a_ref[0], a_ref[1]], axis=1) * nd_ref[...]
        o = o + b_ref[...]
        o = jnp.concatenate([o[:, :1] + f_ref[...][:, 1:2], o[:, 1:]], axis=1)
        o_ref[...] = o

    return pl.pallas_call(
        body,
        grid=(N // NB,),
        in_specs=[
            pl.BlockSpec((2, NB, F_OUT // 2), lambda i: (0, i, 0)),
            pl.BlockSpec((NB, 1), lambda i: (i, 0)),
            pl.BlockSpec((1, F_OUT), lambda i: (0, 0)),
            pl.BlockSpec((NB, F_IN), lambda i: (i, 0)),
        ],
        out_specs=pl.BlockSpec((NB, F_OUT), lambda i: (i, 0)),
        out_shape=jax.ShapeDtypeStruct((N, F_OUT), jnp.float32),
    )(agg, nd, b3, features)


def kernel(features, edge_index, W1, b1, W2, b2, W3, b3):
    edges_r = edge_index.reshape(2, NS, EPT)
    srcr = edge_index[0].reshape(NS, NCHUNK, CHUNK)
    dstr = edge_index[1].reshape(NS, NCHUNK, CHUNK)

    hists = _sc_degree_hists(edges_r)
    ns, nd, xs = _tc_norms_scale(hists, features)
    agg1 = _sc_propagate(xs, srcr, dstr, F_IN // 2)
    h1s = _tc_layer1(agg1, nd, ns, W1, b1.reshape(1, -1))
    agg2 = _sc_propagate(h1s, srcr, dstr, F_HID // 4)
    t = _tc_layer2(agg2, nd, ns, W2, b2.reshape(1, -1), W3)
    agg3 = _sc_propagate(t, srcr, dstr, F_OUT // 2)
    return _tc_final(agg3, nd, b3.reshape(1, -1), features)


# speedup vs baseline: 1.0217x; 1.0217x over previous
import dataclasses
import functools

import jax
import jax.numpy as jnp
from jax import lax
from jax.experimental import pallas as pl
from jax.experimental.pallas import tpu as pltpu
from jax.experimental.pallas import tpu_sc as plsc

N = 10000
E = 320000
F_IN = 128
F_HID = 256
F_OUT = 64

NC = 2
NS = 16
LANES = 16
EPT = E // NS
CHUNK = 125
NCHUNK = EPT // CHUNK
ROWS_PER_TILE = N // NS
NB = 2000
DEPTH = 5


def _sc_compiler_params(untiled=False):
    cp = pltpu.CompilerParams()
    if "needs_layout_passes" in pltpu.CompilerParams.__dataclass_fields__:
        cp = dataclasses.replace(cp, needs_layout_passes=False)
    if untiled and "use_tc_tiling_on_sc" in pltpu.CompilerParams.__dataclass_fields__:
        cp = dataclasses.replace(cp, use_tc_tiling_on_sc=False)
    return cp


def _sc_degree_hists(edges_r):
    mesh = plsc.VectorSubcoreMesh(core_axis_name="c", subcore_axis_name="s")

    @functools.partial(
        pl.kernel,
        out_type=jax.ShapeDtypeStruct((NC, NS, N), jnp.float32),
        mesh=mesh,
        compiler_params=_sc_compiler_params(),
        scratch_types=[
            pltpu.VMEM((EPT,), jnp.int32),
            pltpu.VMEM((N,), jnp.float32),
        ],
    )
    def k(edges_hbm, out_hbm, ebuf, hist):
        c = lax.axis_index("c")
        s = lax.axis_index("s")
        pltpu.sync_copy(edges_hbm.at[c, s], ebuf)
        zeros16 = jnp.zeros((LANES,), jnp.float32)
        ones16 = jnp.ones((LANES,), jnp.float32)

        @pl.loop(0, N // LANES)
        def _(i):
            hist[pl.ds(i * LANES, LANES)] = zeros16

        @pl.loop(0, EPT // LANES)
        def _(i):
            idx = ebuf[pl.ds(i * LANES, LANES)]
            plsc.addupdate_scatter(hist, [idx], ones16)

        pltpu.sync_copy(hist, out_hbm.at[c, s])

    return k(edges_r)


def _sc_propagate(xblocks, srcr, dstr, fb):
    nblk = xblocks.shape[0]
    per_core = nblk // NC
    mesh = plsc.VectorSubcoreMesh(core_axis_name="c", subcore_axis_name="s")

    @functools.partial(
        pl.kernel,
        out_type=jax.ShapeDtypeStruct((nblk, NS, ROWS_PER_TILE, fb),
                                      jnp.float32),
        mesh=mesh,
        compiler_params=_sc_compiler_params(untiled=True),
        scratch_types=(
            [pltpu.VMEM((NCHUNK, CHUNK), jnp.int32),
             pltpu.VMEM((NCHUNK, CHUNK), jnp.int32)]
            + [pltpu.VMEM((CHUNK, fb), jnp.float32)] * (DEPTH + 1)
            + [pltpu.VMEM_SHARED((N, fb), jnp.float32)]
            + [pltpu.SemaphoreType.DMA] * (2 * DEPTH)
        ),
    )
    def k(*refs):
        xb_hbm, src_hbm, dst_hbm, out_hbm, src_v, dst_v = refs[:6]
        gbufs = refs[6:6 + DEPTH]
        zbuf = refs[6 + DEPTH]
        acc = refs[7 + DEPTH]
        sgs = refs[8 + DEPTH:8 + 2 * DEPTH]
        sss = refs[8 + 2 * DEPTH:8 + 3 * DEPTH]
        x_hbms = [xb_hbm.at[b] for b in range(nblk)]
        c = lax.axis_index("c")
        s = lax.axis_index("s")
        zeros16 = jnp.zeros((LANES,), jnp.float32)

        @pl.loop(0, CHUNK)
        def _(r):
            @pl.loop(0, fb // LANES)
            def _(kk):
                zbuf[r, pl.ds(kk * LANES, LANES)] = zeros16

        pltpu.sync_copy(src_hbm.at[s], src_v)
        pltpu.sync_copy(dst_hbm.at[s], dst_v)

        def run_pass(x_hbm, blk):
            for q in range(ROWS_PER_TILE // CHUNK):
                pltpu.sync_copy(
                    zbuf, acc.at[pl.ds(s * ROWS_PER_TILE + q * CHUNK, CHUNK)])
            plsc.subcore_barrier()

            def wait_g(b):
                pltpu.make_async_copy(x_hbm.at[src_v.at[0]], gbufs[b],
                                      sgs[b]).wait()

            def wait_s(b):
                pltpu.make_async_copy(gbufs[b], acc.at[dst_v.at[0]],
                                      sss[b]).wait()

            for b in range(DEPTH // 2):
                pltpu.async_copy(x_hbm.at[src_v.at[b]], gbufs[b], sgs[b])

            @pl.loop(0, NCHUNK // DEPTH)
            def _(jj):
                for r in range(DEPTH):
                    j = DEPTH * jj + r
                    bn = (r + DEPTH // 2) % DEPTH

                    def step1(bn=bn):
                        wait_s(bn)

                    def step2(j=j, bn=bn):
                        pltpu.async_copy(x_hbm.at[src_v.at[j + DEPTH // 2]],
                                         gbufs[bn], sgs[bn])

                    if r < DEPTH - DEPTH // 2:
                        @pl.when(jj > 0)
                        def _(step1=step1):
                            step1()

                        step2()
                    else:
                        step1()

                        @pl.when(jj < NCHUNK // DEPTH - 1)
                        def _(step2=step2):
                            step2()

                    wait_g(r)
                    pltpu.async_copy(gbufs[r], acc.at[dst_v.at[j]], sss[r],
                                     add=True)

            for b in range(DEPTH // 2, DEPTH):
                wait_s(b)
            plsc.subcore_barrier()
            pltpu.sync_copy(acc.at[pl.ds(s * ROWS_PER_TILE, ROWS_PER_TILE)],
                            out_hbm.at[blk, s])

        for p in range(per_core):
            @pl.when(c == 0)
            def _():
                run_pass(x_hbms[p], p)

            @pl.when(c == 1)
            def _():
                run_pass(x_hbms[per_core + p], per_core + p)

    return k(xblocks, srcr, dstr).reshape(nblk, N, fb)


def _tc_norms_scale(hists, features):
    def body(h_ref, f_ref, ns_ref, nd_ref, xs_ref):
        deg = jnp.sum(h_ref[...], axis=1)
        norm = jnp.where(deg > 0, lax.rsqrt(jnp.maximum(deg, 1.0)), 0.0)
        ns = norm[0][:, None]
        nd = norm[1][:, None]
        ns_ref[...] = ns
        nd_ref[...] = nd
        xsc = f_ref[...] * ns
        xs_ref[0] = xsc[:, : F_IN // 2]
        xs_ref[1] = xsc[:, F_IN // 2:]

    return pl.pallas_call(
        body,
        out_shape=[
            jax.ShapeDtypeStruct((N, 1), jnp.float32),
            jax.ShapeDtypeStruct((N, 1), jnp.float32),
            jax.ShapeDtypeStruct((2, N, F_IN // 2), jnp.float32),
        ],
    )(hists, features)


def _tc_layer1(agg, nd, ns, W1, b1):
    def body(a_ref, nd_ref, ns_ref, w_ref, b_ref, o_ref):
        a = jnp.concatenate([a_ref[0], a_ref[1]], axis=1) * nd_ref[...]
        h = jnp.dot(a, w_ref[...], precision=lax.Precision.HIGHEST) + b_ref[...]
        h = jnp.maximum(h, 0.0) * ns_ref[...]
        for b in range(4):
            o_ref[b] = h[:, b * (F_HID // 4): (b + 1) * (F_HID // 4)]

    return pl.pallas_call(
        body,
        grid=(N // NB,),
        in_specs=[
            pl.BlockSpec((2, NB, F_IN // 2), lambda i: (0, i, 0)),
            pl.BlockSpec((NB, 1), lambda i: (i, 0)),
            pl.BlockSpec((NB, 1), lambda i: (i, 0)),
            pl.BlockSpec((F_IN, F_HID), lambda i: (0, 0)),
            pl.BlockSpec((1, F_HID), lambda i: (0, 0)),
        ],
        out_specs=pl.BlockSpec((4, NB, F_HID // 4), lambda i: (0, i, 0)),
        out_shape=jax.ShapeDtypeStruct((4, N, F_HID // 4), jnp.float32),
    )(agg, nd, ns, W1, b1)


def _tc_layer2(agg, nd, ns, W2, b2, W3):
    def body(a_ref, nd_ref, ns_ref, w2_ref, b2_ref, w3_ref, o_ref):
        a = jnp.concatenate([a_ref[b] for b in range(4)], axis=1) * nd_ref[...]
        h = jnp.dot(a, w2_ref[...], precision=lax.Precision.HIGHEST) + b2_ref[...]
        h = jnp.maximum(h, 0.0) * ns_ref[...]
        t = jnp.dot(h, w3_ref[...], precision=lax.Precision.HIGHEST)
        o_ref[0] = t[:, : F_OUT // 2]
        o_ref[1] = t[:, F_OUT // 2:]

    return pl.pallas_call(
        body,
        grid=(N // NB,),
        in_specs=[
            pl.BlockSpec((4, NB, F_HID // 4), lambda i: (0, i, 0)),
            pl.BlockSpec((NB, 1), lambda i: (i, 0)),
            pl.BlockSpec((NB, 1), lambda i: (i, 0)),
            pl.BlockSpec((F_HID, F_HID), lambda i: (0, 0)),
            pl.BlockSpec((1, F_HID), lambda i: (0, 0)),
            pl.BlockSpec((F_HID, F_OUT), lambda i: (0, 0)),
        ],
        out_specs=pl.BlockSpec((2, NB, F_OUT // 2), lambda i: (0, i, 0)),
        out_shape=jax.ShapeDtypeStruct((2, N, F_OUT // 2), jnp.float32),
    )(agg, nd, ns, W2, b2, W3)


def _tc_final(agg, nd, b3, features):
    def body(a_ref, nd_ref, b_ref, f_ref, o_ref):
        o = jnp.concatenate([a_ref[0], a_ref[1]], axis=1) * nd_ref[...]
        o = o + b_ref[...]
        o = jnp.concatenate([o[:, :1] + f_ref[...][:, 1:2], o[:, 1:]], axis=1)
        o_ref[...] = o

    return pl.pallas_call(
        body,
        grid=(N // NB,),
        in_specs=[
            pl.BlockSpec((2, NB, F_OUT // 2), lambda i: (0, i, 0)),
            pl.BlockSpec((NB, 1), lambda i: (i, 0)),
            pl.BlockSpec((1, F_OUT), lambda i: (0, 0)),
            pl.BlockSpec((NB, F_IN), lambda i: (i, 0)),
        ],
        out_specs=pl.BlockSpec((NB, F_OUT), lambda i: (i, 0)),
        out_shape=jax.ShapeDtypeStruct((N, F_OUT), jnp.float32),
    )(agg, nd, b3, features)


def kernel(features, edge_index, W1, b1, W2, b2, W3, b3):
    edges_r = edge_index.reshape(2, NS, EPT)
    srcr = edge_index[0].reshape(NS, NCHUNK, CHUNK)
    dstr = edge_index[1].reshape(NS, NCHUNK, CHUNK)

    hists = _sc_degree_hists(edges_r)
    ns, nd, xs = _tc_norms_scale(hists, features)
    agg1 = _sc_propagate(xs, srcr, dstr, F_IN // 2)
    h1s = _tc_layer1(agg1, nd, ns, W1, b1.reshape(1, -1))
    agg2 = _sc_propagate(h1s, srcr, dstr, F_HID // 4)
    t = _tc_layer2(agg2, nd, ns, W2, b2.reshape(1, -1), W3)
    agg3 = _sc_propagate(t, srcr, dstr, F_OUT // 2)
    return _tc_final(agg3, nd, b3.reshape(1, -1), features)


# speedup vs baseline: 1.1501x; 1.1256x over previous
import dataclasses
import functools

import jax
import jax.numpy as jnp
from jax import lax
from jax.experimental import pallas as pl
from jax.experimental.pallas import tpu as pltpu
from jax.experimental.pallas import tpu_sc as plsc

N = 10000
E = 320000
F_IN = 128
F_HID = 256
F_OUT = 64

NC = 2
NS = 16
LANES = 16
EPT = E // NS
CHUNK = 125
NCHUNK = EPT // CHUNK
ROWS_PER_TILE = N // NS
NB = 2000
DEPTH = 5


def _sc_compiler_params(untiled=False):
    cp = pltpu.CompilerParams()
    if "needs_layout_passes" in pltpu.CompilerParams.__dataclass_fields__:
        cp = dataclasses.replace(cp, needs_layout_passes=False)
    if untiled and "use_tc_tiling_on_sc" in pltpu.CompilerParams.__dataclass_fields__:
        cp = dataclasses.replace(cp, use_tc_tiling_on_sc=False)
    return cp


def _sc_degree_hists(edges_r):
    mesh = plsc.VectorSubcoreMesh(core_axis_name="c", subcore_axis_name="s")

    @functools.partial(
        pl.kernel,
        out_type=jax.ShapeDtypeStruct((NC, NS, N), jnp.float32),
        mesh=mesh,
        compiler_params=_sc_compiler_params(),
        scratch_types=[
            pltpu.VMEM((EPT,), jnp.int32),
            pltpu.VMEM((N,), jnp.float32),
        ],
    )
    def k(edges_hbm, out_hbm, ebuf, hist):
        c = lax.axis_index("c")
        s = lax.axis_index("s")
        pltpu.sync_copy(edges_hbm.at[c, s], ebuf)
        zeros16 = jnp.zeros((LANES,), jnp.float32)
        ones16 = jnp.ones((LANES,), jnp.float32)

        @pl.loop(0, N // LANES)
        def _(i):
            hist[pl.ds(i * LANES, LANES)] = zeros16

        @pl.loop(0, EPT // LANES)
        def _(i):
            idx = ebuf[pl.ds(i * LANES, LANES)]
            plsc.addupdate_scatter(hist, [idx], ones16)

        pltpu.sync_copy(hist, out_hbm.at[c, s])

    return k(edges_r)


def _sc_propagate(xblocks, srcr, dstr, fb, bspec, n_out):
    nblk = len(bspec)
    per_core = nblk // NC
    mesh = plsc.VectorSubcoreMesh(core_axis_name="c", subcore_axis_name="s")

    @functools.partial(
        pl.kernel,
        out_type=[jax.ShapeDtypeStruct((N, 128), jnp.float32)] * n_out,
        mesh=mesh,
        compiler_params=_sc_compiler_params(untiled=True),
        scratch_types=(
            [pltpu.VMEM((NCHUNK, CHUNK), jnp.int32),
             pltpu.VMEM((NCHUNK, CHUNK), jnp.int32)]
            + [pltpu.VMEM((CHUNK, fb), jnp.float32)] * (DEPTH + 1)
            + [pltpu.VMEM_SHARED((N, fb), jnp.float32)]
            + [pltpu.SemaphoreType.DMA] * (2 * DEPTH)
        ),
    )
    def k(*refs):
        xb_hbm, src_hbm, dst_hbm = refs[:3]
        outs = refs[3:3 + n_out]
        rest = refs[3 + n_out:]
        src_v, dst_v = rest[:2]
        gbufs = rest[2:2 + DEPTH]
        zbuf = rest[2 + DEPTH]
        acc = rest[3 + DEPTH]
        sgs = rest[4 + DEPTH:4 + 2 * DEPTH]
        sss = rest[4 + 2 * DEPTH:4 + 3 * DEPTH]
        c = lax.axis_index("c")
        s = lax.axis_index("s")
        zeros16 = jnp.zeros((LANES,), jnp.float32)

        @pl.loop(0, CHUNK)
        def _(r):
            @pl.loop(0, fb // LANES)
            def _(kk):
                zbuf[r, pl.ds(kk * LANES, LANES)] = zeros16

        pltpu.sync_copy(src_hbm.at[s], src_v)
        pltpu.sync_copy(dst_hbm.at[s], dst_v)

        def run_pass(x_hbm, out_ref, out_col):
            for q in range(ROWS_PER_TILE // CHUNK):
                pltpu.sync_copy(
                    zbuf, acc.at[pl.ds(s * ROWS_PER_TILE + q * CHUNK, CHUNK)])
            plsc.subcore_barrier()

            def gather_src(j):
                return x_hbm.at[src_v.at[j]]

            def wait_g(b):
                pltpu.make_async_copy(gather_src(0), gbufs[b],
                                      sgs[b]).wait()

            def wait_s(b):
                pltpu.make_async_copy(gbufs[b], acc.at[dst_v.at[0]],
                                      sss[b]).wait()

            for b in range(DEPTH // 2):
                pltpu.async_copy(gather_src(b), gbufs[b], sgs[b])

            @pl.loop(0, NCHUNK // DEPTH)
            def _(jj):
                for r in range(DEPTH):
                    j = DEPTH * jj + r
                    bn = (r + DEPTH // 2) % DEPTH

                    def step1(bn=bn):
                        wait_s(bn)

                    def step2(j=j, bn=bn):
                        pltpu.async_copy(gather_src(j + DEPTH // 2),
                                         gbufs[bn], sgs[bn])

                    if r < DEPTH - DEPTH // 2:
                        @pl.when(jj > 0)
                        def _(step1=step1):
                            step1()

                        step2()
                    else:
                        step1()

                        @pl.when(jj < NCHUNK // DEPTH - 1)
                        def _(step2=step2):
                            step2()

                    wait_g(r)
                    pltpu.async_copy(gbufs[r], acc.at[dst_v.at[j]], sss[r],
                                     add=True)

            for b in range(DEPTH // 2, DEPTH):
                wait_s(b)
            plsc.subcore_barrier()
            pltpu.sync_copy(
                acc.at[pl.ds(s * ROWS_PER_TILE, ROWS_PER_TILE)],
                out_ref.at[pl.ds(s * ROWS_PER_TILE, ROWS_PER_TILE),
                           pl.ds(out_col, fb)])

        for p in range(per_core):
            for cv in range(NC):
                b = cv * per_core + p
                oa, ocol = bspec[b]

                @pl.when(c == cv)
                def _(b=b, oa=oa, ocol=ocol):
                    run_pass(xb_hbm.at[b], outs[oa], ocol)

    res = k(xblocks, srcr, dstr)
    if not isinstance(res, (list, tuple)):
        res = [res]
    return res


def _tc_norms_scale(hists, features):
    def body(h_ref, f_ref, ns_ref, nd_ref, xs_ref):
        deg = jnp.sum(h_ref[...], axis=1)
        norm = jnp.where(deg > 0, lax.rsqrt(jnp.maximum(deg, 1.0)), 0.0)
        ns = norm[0][:, None]
        nd = norm[1][:, None]
        ns_ref[...] = ns
        nd_ref[...] = nd
        xsc = f_ref[...] * ns
        xs_ref[0] = xsc[:, : F_IN // 2]
        xs_ref[1] = xsc[:, F_IN // 2:]

    return pl.pallas_call(
        body,
        out_shape=[
            jax.ShapeDtypeStruct((N, 1), jnp.float32),
            jax.ShapeDtypeStruct((N, 1), jnp.float32),
            jax.ShapeDtypeStruct((2, N, F_IN // 2), jnp.float32),
        ],
    )(hists, features)


def _tc_layer1(agg, nd, ns, W1, b1):
    def body(a_ref, nd_ref, ns_ref, w_ref, b_ref, o_ref):
        a = a_ref[...] * nd_ref[...]
        h = jnp.dot(a, w_ref[...]) + b_ref[...]
        h = jnp.maximum(h, 0.0) * ns_ref[...]
        for b in range(4):
            o_ref[b] = h[:, b * (F_HID // 4): (b + 1) * (F_HID // 4)]

    return pl.pallas_call(
        body,
        grid=(N // NB,),
        in_specs=[
            pl.BlockSpec((NB, F_IN), lambda i: (i, 0)),
            pl.BlockSpec((NB, 1), lambda i: (i, 0)),
            pl.BlockSpec((NB, 1), lambda i: (i, 0)),
            pl.BlockSpec((F_IN, F_HID), lambda i: (0, 0)),
            pl.BlockSpec((1, F_HID), lambda i: (0, 0)),
        ],
        out_specs=pl.BlockSpec((4, NB, F_HID // 4), lambda i: (0, i, 0)),
        out_shape=jax.ShapeDtypeStruct((4, N, F_HID // 4), jnp.float32),
    )(agg, nd, ns, W1, b1)


def _tc_layer2(agg_a, agg_b, nd, ns, W2, b2, W3):
    def body(aa_ref, ab_ref, nd_ref, ns_ref, w2_ref, b2_ref, w3_ref, o_ref):
        a = jnp.concatenate([aa_ref[...], ab_ref[...]], axis=1) * nd_ref[...]
        h = jnp.dot(a, w2_ref[...]) + b2_ref[...]
        h = jnp.maximum(h, 0.0) * ns_ref[...]
        t = jnp.dot(h, w3_ref[...])
        o_ref[0] = t[:, : F_OUT // 2]
        o_ref[1] = t[:, F_OUT // 2:]

    return pl.pallas_call(
        body,
        grid=(N // NB,),
        in_specs=[
            pl.BlockSpec((NB, F_HID // 2), lambda i: (i, 0)),
            pl.BlockSpec((NB, F_HID // 2), lambda i: (i, 0)),
            pl.BlockSpec((NB, 1), lambda i: (i, 0)),
            pl.BlockSpec((NB, 1), lambda i: (i, 0)),
            pl.BlockSpec((F_HID, F_HID), lambda i: (0, 0)),
            pl.BlockSpec((1, F_HID), lambda i: (0, 0)),
            pl.BlockSpec((F_HID, F_OUT), lambda i: (0, 0)),
        ],
        out_specs=pl.BlockSpec((2, NB, F_OUT // 2), lambda i: (0, i, 0)),
        out_shape=jax.ShapeDtypeStruct((2, N, F_OUT // 2), jnp.float32),
    )(agg_a, agg_b, nd, ns, W2, b2, W3)


def _tc_final(agg, nd, b3, features):
    def body(a_ref, nd_ref, b_ref, f_ref, o_ref):
        o = a_ref[...][:, :F_OUT] * nd_ref[...] + b_ref[...]
        o = jnp.concatenate([o[:, :1] + f_ref[...][:, 1:2], o[:, 1:]], axis=1)
        o_ref[...] = o

    return pl.pallas_call(
        body,
        grid=(N // NB,),
        in_specs=[
            pl.BlockSpec((NB, 2 * F_OUT), lambda i: (i, 0)),
            pl.BlockSpec((NB, 1), lambda i: (i, 0)),
            pl.BlockSpec((1, F_OUT), lambda i: (0, 0)),
            pl.BlockSpec((NB, F_IN), lambda i: (i, 0)),
        ],
        out_specs=pl.BlockSpec((NB, F_OUT), lambda i: (i, 0)),
        out_shape=jax.ShapeDtypeStruct((N, F_OUT), jnp.float32),
    )(agg, nd, b3, features)


def kernel(features, edge_index, W1, b1, W2, b2, W3, b3):
    edges_r = edge_index.reshape(2, NS, EPT)
    srcr = edge_index[0].reshape(NS, NCHUNK, CHUNK)
    dstr = edge_index[1].reshape(NS, NCHUNK, CHUNK)

    hists = _sc_degree_hists(edges_r)
    ns, nd, xs = _tc_norms_scale(hists, features)
    agg1, = _sc_propagate(xs, srcr, dstr, 64, [(0, 0), (0, 64)], 1)
    h1s = _tc_layer1(agg1, nd, ns, W1, b1.reshape(1, -1))
    agg2a, agg2b = _sc_propagate(h1s, srcr, dstr, 64,
                                 [(0, 0), (0, 64), (1, 0), (1, 64)], 2)
    t = _tc_layer2(agg2a, agg2b, nd, ns, W2, b2.reshape(1, -1), W3)
    agg3, = _sc_propagate(t, srcr, dstr, 32, [(0, 0), (0, 32)], 1)
    return _tc_final(agg3, nd, b3.reshape(1, -1), features)
